# Initial kernel scaffold; baseline (speedup 1.0000x reference)
import jax
import jax.numpy as jnp
from jax.experimental import pallas as pl


def kernel(x, edge_index, edge_pairs, relation, bias):
    raise NotImplementedError("write your pallas kernel here")


# speedup vs baseline: 4.7971x; 4.7971x over previous
import jax
import jax.numpy as jnp
from jax import lax
from jax.experimental import pallas as pl
from jax.experimental.pallas import tpu as pltpu
from jax.experimental.pallas import tpu_sc as plsc

N_NODES = 10000
N_EDGES = 320000
D = 128
L = 16

NC = 2
NS = 16
NW = NC * NS
EPW = N_EDGES // NW
CHUNK = 400
N_CHUNKS = EPW // CHUNK
N_GROUPS = CHUNK // L


def _sc_body(x_hbm, src_hbm, dst_hbm, rel_hbm, bias_hbm, out_hbm,
             src_v, dst_v, xu_v, xv_v, out_v, rel_v, bias_v, acc_s,
             sem_u, sem_v):
    wid = lax.axis_index("s") * NC + lax.axis_index("c")
    base = pl.multiple_of(wid * EPW, 8)

    pltpu.sync_copy(rel_hbm, rel_v)
    pltpu.sync_copy(bias_hbm, bias_v)
    rel = [rel_v[pl.ds(i * L, L)] for i in range(D // L)]
    bias_vec = bias_v[pl.ds(0, L)]
    lane = lax.iota(jnp.int32, L)

    def chunk_body(c, carry):
        cb = pl.multiple_of(base + c * CHUNK, 8)
        pltpu.sync_copy(src_hbm.at[pl.ds(cb, CHUNK)], src_v)
        pltpu.sync_copy(dst_hbm.at[pl.ds(cb, CHUNK)], dst_v)
        cp_u = pltpu.async_copy(x_hbm.at[src_v], xu_v, sem_u)
        cp_v = pltpu.async_copy(x_hbm.at[dst_v], xv_v, sem_v)
        cp_u.wait()
        cp_v.wait()

        def group_body(g, gcarry):
            def edge_body(j, ecarry):
                e = g * L + j
                acc = xu_v[e, pl.ds(0, L)] * (rel[0] * xv_v[e, pl.ds(0, L)])
                for i in range(1, D // L):
                    acc = acc + xu_v[e, pl.ds(i * L, L)] * (
                        rel[i] * xv_v[e, pl.ds(i * L, L)])
                plsc.store_scatter(acc_s, [lane * L + j], acc)
                return ecarry

            lax.fori_loop(0, L, edge_body, 0)
            tot = acc_s[pl.ds(0, L)]
            for i in range(1, L):
                tot = tot + acc_s[pl.ds(i * L, L)]
            out_v[pl.ds(g * L, L)] = tot + bias_vec
            return gcarry

        lax.fori_loop(0, N_GROUPS, group_body, 0)
        pltpu.sync_copy(out_v, out_hbm.at[pl.ds(cb, CHUNK)])
        return carry

    lax.fori_loop(0, N_CHUNKS, chunk_body, 0)


@jax.jit
def _scores_sc(x, src, dst, relation, bias16):
    mesh = plsc.VectorSubcoreMesh(core_axis_name="c", subcore_axis_name="s")
    return pl.kernel(
        _sc_body,
        out_type=jax.ShapeDtypeStruct((N_EDGES,), jnp.float32),
        mesh=mesh,
        scratch_types=[
            pltpu.VMEM((CHUNK,), jnp.int32),
            pltpu.VMEM((CHUNK,), jnp.int32),
            pltpu.VMEM((CHUNK, D), jnp.float32),
            pltpu.VMEM((CHUNK, D), jnp.float32),
            pltpu.VMEM((CHUNK,), jnp.float32),
            pltpu.VMEM((D,), jnp.float32),
            pltpu.VMEM((L,), jnp.float32),
            pltpu.VMEM((L * L,), jnp.float32),
            pltpu.SemaphoreType.DMA,
            pltpu.SemaphoreType.DMA,
        ],
        compiler_params=pltpu.CompilerParams(needs_layout_passes=False),
        name="distmult_sc",
    )(x, src, dst, relation, bias16)


def kernel(x, edge_index, edge_pairs, relation, bias):
    del edge_index
    ep = edge_pairs.astype(jnp.int32)
    src = ep[:, 0]
    dst = ep[:, 1]
    bias16 = jnp.broadcast_to(bias.astype(jnp.float32), (L,))
    return _scores_sc(x, src, dst, relation.astype(jnp.float32), bias16)


# speedup vs baseline: 8.2247x; 1.7145x over previous
import jax
import jax.numpy as jnp
from jax import lax
from jax.experimental import pallas as pl
from jax.experimental.pallas import tpu as pltpu
from jax.experimental.pallas import tpu_sc as plsc

N_NODES = 10000
N_EDGES = 320000
D = 128
L = 16

NC = 2
NS = 16
NW = NC * NS
EPW = N_EDGES // NW
CHUNK = 200
N_CHUNKS = EPW // CHUNK
N_GROUPS = CHUNK // L
REM = CHUNK - N_GROUPS * L


def _sc_body(x_hbm, src_hbm, dst_hbm, rel_hbm, bias_hbm, out_hbm,
             src_v, dst_v, xu0, xv0, xu1, xv1, out0, out1,
             rel_v, bias_v, acc_s,
             sem_u0, sem_v0, sem_u1, sem_v1, sem_o0, sem_o1):
    wid = lax.axis_index("s") * NC + lax.axis_index("c")
    base = pl.multiple_of(wid * EPW, 8)

    pltpu.sync_copy(rel_hbm, rel_v)
    pltpu.sync_copy(bias_hbm, bias_v)
    pltpu.sync_copy(src_hbm.at[pl.ds(base, EPW)], src_v)
    pltpu.sync_copy(dst_hbm.at[pl.ds(base, EPW)], dst_v)
    rel = [rel_v[pl.ds(i * L, L)] for i in range(D // L)]
    bias_vec = bias_v[pl.ds(0, L)]
    lane = lax.iota(jnp.int32, L)

    bufs = ((xu0, xv0, out0, sem_u0, sem_v0, sem_o0),
            (xu1, xv1, out1, sem_u1, sem_v1, sem_o1))

    def issue(c, xu, xv, sem_u, sem_v):
        off = c * CHUNK
        pltpu.async_copy(x_hbm.at[src_v.at[pl.ds(off, CHUNK)]], xu, sem_u)
        pltpu.async_copy(x_hbm.at[dst_v.at[pl.ds(off, CHUNK)]], xv, sem_v)

    def wait_rows(xu, xv, sem_u, sem_v):
        pltpu.make_async_copy(x_hbm.at[pl.ds(0, CHUNK)], xu, sem_u).wait()
        pltpu.make_async_copy(x_hbm.at[pl.ds(0, CHUNK)], xv, sem_v).wait()

    def compute(xu, xv, out_v):
        def group_body(g, gcarry):
            def edge_body(j, ecarry):
                e = g * L + j
                acc = xu[e, pl.ds(0, L)] * (rel[0] * xv[e, pl.ds(0, L)])
                for i in range(1, D // L):
                    acc = acc + xu[e, pl.ds(i * L, L)] * (
                        rel[i] * xv[e, pl.ds(i * L, L)])
                plsc.store_scatter(acc_s, [lane * L + j], acc)
                return ecarry

            lax.fori_loop(0, L, edge_body, 0)
            tot = acc_s[pl.ds(0, L)]
            for i in range(1, L):
                tot = tot + acc_s[pl.ds(i * L, L)]
            out_v[pl.ds(g * L, L)] = tot + bias_vec
            return gcarry

        lax.fori_loop(0, N_GROUPS, group_body, 0)
        if REM:
            def tail_edge(j, ecarry):
                e = N_GROUPS * L + j
                acc = xu[e, pl.ds(0, L)] * (rel[0] * xv[e, pl.ds(0, L)])
                for i in range(1, D // L):
                    acc = acc + xu[e, pl.ds(i * L, L)] * (
                        rel[i] * xv[e, pl.ds(i * L, L)])
                plsc.store_scatter(acc_s, [lane * L + j], acc)
                return ecarry

            lax.fori_loop(0, REM, tail_edge, 0)
            tot = acc_s[pl.ds(0, L)]
            for i in range(1, L):
                tot = tot + acc_s[pl.ds(i * L, L)]
            mask = lane < REM
            plsc.store_scatter(out_v, [N_GROUPS * L + lane], tot + bias_vec,
                               mask=mask)

    issue(0, xu0, xv0, sem_u0, sem_v0)
    issue(1, xu1, xv1, sem_u1, sem_v1)

    def pair_body(p, carry):
        for s in range(2):
            c = 2 * p + s
            xu, xv, out_v, sem_u, sem_v, sem_o = bufs[s]
            wait_rows(xu, xv, sem_u, sem_v)

            @pl.when(c >= 2)
            def _():
                pltpu.make_async_copy(
                    out_v, out_hbm.at[pl.ds(0, CHUNK)], sem_o).wait()

            compute(xu, xv, out_v)

            @pl.when(c + 2 < N_CHUNKS)
            def _():
                issue(c + 2, xu, xv, sem_u, sem_v)

            cb = pl.multiple_of(base + c * CHUNK, 8)
            pltpu.async_copy(out_v, out_hbm.at[pl.ds(cb, CHUNK)], sem_o)
        return carry

    lax.fori_loop(0, N_CHUNKS // 2, pair_body, 0)
    pltpu.make_async_copy(out0, out_hbm.at[pl.ds(0, CHUNK)], sem_o0).wait()
    pltpu.make_async_copy(out1, out_hbm.at[pl.ds(0, CHUNK)], sem_o1).wait()


@jax.jit
def _scores_sc(x, src, dst, relation, bias16):
    mesh = plsc.VectorSubcoreMesh(core_axis_name="c", subcore_axis_name="s")
    return pl.kernel(
        _sc_body,
        out_type=jax.ShapeDtypeStruct((N_EDGES,), jnp.float32),
        mesh=mesh,
        scratch_types=[
            pltpu.VMEM((EPW,), jnp.int32),
            pltpu.VMEM((EPW,), jnp.int32),
            pltpu.VMEM((CHUNK, D), jnp.float32),
            pltpu.VMEM((CHUNK, D), jnp.float32),
            pltpu.VMEM((CHUNK, D), jnp.float32),
            pltpu.VMEM((CHUNK, D), jnp.float32),
            pltpu.VMEM((CHUNK,), jnp.float32),
            pltpu.VMEM((CHUNK,), jnp.float32),
            pltpu.VMEM((D,), jnp.float32),
            pltpu.VMEM((L,), jnp.float32),
            pltpu.VMEM((L * L,), jnp.float32),
            pltpu.SemaphoreType.DMA,
            pltpu.SemaphoreType.DMA,
            pltpu.SemaphoreType.DMA,
            pltpu.SemaphoreType.DMA,
            pltpu.SemaphoreType.DMA,
            pltpu.SemaphoreType.DMA,
        ],
        compiler_params=pltpu.CompilerParams(needs_layout_passes=False),
        name="distmult_sc",
    )(x, src, dst, relation, bias16)


def kernel(x, edge_index, edge_pairs, relation, bias):
    del edge_index
    ep = edge_pairs.astype(jnp.int32)
    src = ep[:, 0]
    dst = ep[:, 1]
    bias16 = jnp.broadcast_to(bias.astype(jnp.float32), (L,))
    return _scores_sc(x, src, dst, relation.astype(jnp.float32), bias16)
